# Initial kernel scaffold; baseline (speedup 1.0000x reference)
import jax
import jax.numpy as jnp
from jax.experimental import pallas as pl


def kernel(x, edge_index, batch, W0, b0, W1, b1, W2, b2, W3, b3, Wl, bl, Wo, bo):
    raise NotImplementedError("write your pallas kernel here")


# speedup vs baseline: 2.3359x; 2.3359x over previous
import functools

import jax
import jax.numpy as jnp
from jax import lax
from jax.experimental import pallas as pl
from jax.experimental.pallas import tpu as pltpu
from jax.experimental.pallas import tpu_sc as plsc

NN = 100000
EE = 1600000
GG = 128
HID = 128

NC, NS, LL = 2, 16, 16
SL = HID // LL
NPAD = 100352
NSTRIPE = NPAD // NS
CH = 128
NCHUNK = EE // CH
CTRIP = (NCHUNK + NS - 1) // NS

PSTRIPE = NPAD // (NC * NS)
MROWS = (GG + 1) * HID
CROWS = (GG + 1) * LL + 112

_mesh = plsc.VectorSubcoreMesh(core_axis_name="c", subcore_axis_name="s")


def _iota16():
    return lax.broadcasted_iota(jnp.int32, (LL,), 0)


@functools.partial(
    pl.kernel,
    out_type=[
        jax.ShapeDtypeStruct((NPAD,), jnp.float32),
        jax.ShapeDtypeStruct((NPAD,), jnp.float32),
    ],
    mesh=_mesh,
    scratch_types=[
        pltpu.VMEM_SHARED((NPAD,), jnp.float32),
        pltpu.VMEM((CH,), jnp.int32),
        pltpu.VMEM((CH,), jnp.float32),
    ],
)
def _deg_kernel(dst_hbm, zer_hbm, deg0_hbm, deg1_hbm, acc_sp, dst_v, ones_v):
    c = lax.axis_index("c")
    s = lax.axis_index("s")
    for j in range(CH // LL):
        ones_v[pl.ds(j * LL, LL)] = jnp.ones((LL,), jnp.float32)
    pltpu.sync_copy(zer_hbm.at[pl.ds(0, NSTRIPE)],
                    acc_sp.at[pl.ds(s * NSTRIPE, NSTRIPE)])
    plsc.subcore_barrier()

    def chunk(t, carry):
        k = c + NC * (s + NS * t)
        @pl.when(k < NCHUNK)
        def _():
            pltpu.sync_copy(dst_hbm.at[pl.ds(k * CH, CH)], dst_v)
            pltpu.sync_copy(ones_v, acc_sp.at[dst_v], add=True)
        return carry

    lax.fori_loop(0, (NCHUNK + NC * NS - 1) // (NC * NS), chunk, jnp.int32(0))
    plsc.subcore_barrier()

    @pl.when(c == 0)
    def _():
        pltpu.sync_copy(acc_sp.at[pl.ds(s * NSTRIPE, NSTRIPE)],
                        deg0_hbm.at[pl.ds(s * NSTRIPE, NSTRIPE)])

    @pl.when(c == 1)
    def _():
        pltpu.sync_copy(acc_sp.at[pl.ds(s * NSTRIPE, NSTRIPE)],
                        deg1_hbm.at[pl.ds(s * NSTRIPE, NSTRIPE)])


@functools.partial(
    pl.kernel,
    out_type=jax.ShapeDtypeStruct((SL, NPAD, LL), jnp.float32),
    mesh=_mesh,
    scratch_types=[
        pltpu.VMEM_SHARED((NPAD, LL), jnp.float32),
        pltpu.VMEM((CH,), jnp.int32),
        pltpu.VMEM((CH,), jnp.int32),
        pltpu.VMEM((CH,), jnp.int32),
        pltpu.VMEM((CH, LL), jnp.float32),
        pltpu.SemaphoreType.DMA,
    ],
    compiler_params=pltpu.CompilerParams(use_tc_tiling_on_sc=False),
)
def _mp_kernel(ysc2_hbm, src_hbm, dst_hbm, zer2_hbm, out_hbm, acc_sp,
               src_v, sidx_v, dst_v, rows_v, sem):
    c = lax.axis_index("c")
    s = lax.axis_index("s")
    for p in range(SL):
        @pl.when(c == p % 2)
        def _():
            pltpu.sync_copy(zer2_hbm,
                            acc_sp.at[pl.ds(s * NSTRIPE, NSTRIPE)])
            plsc.subcore_barrier()

            def chunk(t, carry):
                k = s + NS * t
                @pl.when(k < NCHUNK)
                def _():
                    off = k * CH
                    pltpu.sync_copy(src_hbm.at[pl.ds(off, CH)], src_v)
                    pltpu.sync_copy(dst_hbm.at[pl.ds(off, CH)], dst_v)
                    for j in range(CH // LL):
                        sv = src_v[pl.ds(j * LL, LL)]
                        sidx_v[pl.ds(j * LL, LL)] = sv * SL + p
                    pltpu.async_copy(ysc2_hbm.at[sidx_v], rows_v, sem).wait()
                    pltpu.sync_copy(rows_v, acc_sp.at[dst_v], add=True)
                return carry

            lax.fori_loop(0, CTRIP, chunk, jnp.int32(0))
            plsc.subcore_barrier()
            pltpu.sync_copy(acc_sp.at[pl.ds(s * NSTRIPE, NSTRIPE)],
                            out_hbm.at[p, pl.ds(s * NSTRIPE, NSTRIPE)])
            plsc.subcore_barrier()


@functools.partial(
    pl.kernel,
    out_type=[
        jax.ShapeDtypeStruct((NC * NS, MROWS), jnp.float32),
        jax.ShapeDtypeStruct((NC * NS, MROWS), jnp.float32),
        jax.ShapeDtypeStruct((NC * NS, CROWS), jnp.float32),
    ],
    mesh=_mesh,
    scratch_types=[
        pltpu.VMEM((MROWS,), jnp.float32),
        pltpu.VMEM((MROWS,), jnp.float32),
        pltpu.VMEM((CROWS,), jnp.float32),
        pltpu.VMEM((LL,), jnp.int32),
        pltpu.VMEM((LL, HID), jnp.float32),
    ],
    compiler_params=pltpu.CompilerParams(needs_layout_passes=False),
)
def _pool_kernel(h_hbm, batch_hbm, mneg_hbm, zer_hbm, maxout, sumout, cntout,
                 macc, sacc, cacc, b_v, rows_v):
    c = lax.axis_index("c")
    s = lax.axis_index("s")
    w = s * NC + c
    pltpu.sync_copy(mneg_hbm, macc)
    pltpu.sync_copy(zer_hbm, sacc)
    pltpu.sync_copy(zer_hbm.at[pl.ds(0, CROWS)], cacc)
    base = w * PSTRIPE
    ones = jnp.ones((LL,), jnp.float32)
    iot = _iota16()

    def step(t, carry):
        off = base + t * LL
        pltpu.sync_copy(batch_hbm.at[pl.ds(off, LL)], b_v)
        pltpu.sync_copy(h_hbm.at[pl.ds(off, LL)], rows_v)
        bv = b_v[...]
        plsc.addupdate_scatter(cacc, [bv * LL + iot], ones)
        for j in range(LL):
            bvj = plsc.load_gather(b_v, [jnp.full((LL,), j, jnp.int32)])
            for q in range(HID // LL):
                addr = bvj * HID + (q * LL + iot)
                r = rows_v[j, pl.ds(q * LL, LL)]
                m = plsc.load_gather(macc, [addr])
                plsc.store_scatter(macc, [addr], jnp.maximum(m, r))
                plsc.addupdate_scatter(sacc, [addr], r)
        return carry

    lax.fori_loop(0, PSTRIPE // LL, step, jnp.int32(0))
    pltpu.sync_copy(macc, maxout.at[w])
    pltpu.sync_copy(sacc, sumout.at[w])
    pltpu.sync_copy(cacc, cntout.at[w])


_BLK = 1024


def _tc0_body(x_ref, d0_ref, d1_ref, w_ref, ysc_ref, dinv_ref):
    d = lax.rsqrt(d0_ref[...] + d1_ref[...] + 1.0)
    y = jnp.dot(x_ref[...], w_ref[...], preferred_element_type=jnp.float32)
    ysc_ref[...] = y * d
    dinv_ref[...] = d


def _tc0(xp, deg0, deg1, W0):
    return pl.pallas_call(
        _tc0_body,
        grid=(NPAD // _BLK,),
        in_specs=[
            pl.BlockSpec((_BLK, 4), lambda i: (i, 0)),
            pl.BlockSpec((_BLK, 1), lambda i: (i, 0)),
            pl.BlockSpec((_BLK, 1), lambda i: (i, 0)),
            pl.BlockSpec((4, HID), lambda i: (0, 0)),
        ],
        out_specs=[
            pl.BlockSpec((_BLK, HID), lambda i: (i, 0)),
            pl.BlockSpec((_BLK, 1), lambda i: (i, 0)),
        ],
        out_shape=[
            jax.ShapeDtypeStruct((NPAD, HID), jnp.float32),
            jax.ShapeDtypeStruct((NPAD, 1), jnp.float32),
        ],
    )(xp, deg0, deg1, W0)


def _acc_concat(acc_ref):
    return jnp.concatenate([acc_ref[p] for p in range(SL)], axis=1)


def _tc_layer_body(acc_ref, ysc_ref, dinv_ref, b_ref, w_ref, out_ref):
    d = dinv_ref[...]
    agg = _acc_concat(acc_ref) + ysc_ref[...]
    h = jnp.tanh(agg * d + b_ref[...][None, :])
    y = jnp.dot(h, w_ref[...], preferred_element_type=jnp.float32)
    out_ref[...] = y * d


def _tc_layer(acc, ysc, dinv, b, W):
    return pl.pallas_call(
        _tc_layer_body,
        grid=(NPAD // _BLK,),
        in_specs=[
            pl.BlockSpec((SL, _BLK, LL), lambda i: (0, i, 0)),
            pl.BlockSpec((_BLK, HID), lambda i: (i, 0)),
            pl.BlockSpec((_BLK, 1), lambda i: (i, 0)),
            pl.BlockSpec((HID,), lambda i: (0,)),
            pl.BlockSpec((HID, HID), lambda i: (0, 0)),
        ],
        out_specs=pl.BlockSpec((_BLK, HID), lambda i: (i, 0)),
        out_shape=jax.ShapeDtypeStruct((NPAD, HID), jnp.float32),
    )(acc, ysc, dinv, b, W)


def _tc4_body(acc_ref, ysc_ref, dinv_ref, b_ref, out_ref):
    agg = _acc_concat(acc_ref) + ysc_ref[...]
    out_ref[...] = jnp.tanh(agg * dinv_ref[...] + b_ref[...][None, :])


def _tc4(acc, ysc, dinv, b):
    return pl.pallas_call(
        _tc4_body,
        grid=(NPAD // _BLK,),
        in_specs=[
            pl.BlockSpec((SL, _BLK, LL), lambda i: (0, i, 0)),
            pl.BlockSpec((_BLK, HID), lambda i: (i, 0)),
            pl.BlockSpec((_BLK, 1), lambda i: (i, 0)),
            pl.BlockSpec((HID,), lambda i: (0,)),
        ],
        out_specs=pl.BlockSpec((_BLK, HID), lambda i: (i, 0)),
        out_shape=jax.ShapeDtypeStruct((NPAD, HID), jnp.float32),
    )(acc, ysc, dinv, b)


_OUTW = 2145 * 2


def _head_body(maxp_ref, sump_ref, cntp_ref, wl_ref, bl_ref, wo_ref, bo_ref,
               out_ref, pooled_ref, cnt_ref):
    maxp = maxp_ref[...]
    sump = sump_ref[...]
    cntp = cntp_ref[...]
    for g in range(GG):
        mg = jnp.max(maxp[:, g * HID:(g + 1) * HID], axis=0)
        sg = jnp.sum(sump[:, g * HID:(g + 1) * HID], axis=0)
        cg = jnp.sum(cntp[:, g * LL:(g + 1) * LL], axis=0)
        pooled_ref[g] = jnp.concatenate([mg, sg])
        cnt_ref[g] = cg
    pm = pooled_ref[...]
    cv = jnp.sum(cnt_ref[...], axis=1, keepdims=True)
    fmax = jnp.where(cv > 0.0, 1.0, 0.0)
    fmean = 1.0 / jnp.maximum(cv, 1.0)
    pooled = jnp.concatenate(
        [pm[:, :HID] * fmax, pm[:, HID:] * fmean], axis=1)
    hid = jnp.dot(pooled, wl_ref[...],
                  preferred_element_type=jnp.float32) + bl_ref[...][None, :]
    out_ref[...] = jnp.dot(hid, wo_ref[...],
                           preferred_element_type=jnp.float32) + bo_ref[...][None, :]


def _head(maxp, sump, cntp, Wl, bl, Wo, bo):
    return pl.pallas_call(
        _head_body,
        out_shape=jax.ShapeDtypeStruct((GG, _OUTW), jnp.float32),
        scratch_shapes=[
            pltpu.VMEM((GG, 2 * HID), jnp.float32),
            pltpu.VMEM((GG, LL), jnp.float32),
        ],
    )(maxp, sump, cntp, Wl, bl, Wo, bo)


def kernel(x, edge_index, batch, W0, b0, W1, b1, W2, b2, W3, b3,
           Wl, bl, Wo, bo):
    src = edge_index[0].astype(jnp.int32)
    dst = edge_index[1].astype(jnp.int32)

    xp = jnp.pad(x, ((0, NPAD - NN), (0, 0)))
    batch_p = jnp.pad(batch.astype(jnp.int32), (0, NPAD - NN),
                      constant_values=GG)

    zer_n = jnp.zeros((NSTRIPE,), jnp.float32)
    zer2 = jnp.zeros((NSTRIPE, LL), jnp.float32)
    deg0, deg1 = _deg_kernel(dst, zer_n)

    ysc, dinv = _tc0(xp, deg0[:, None], deg1[:, None], W0)
    acc = _mp_kernel(ysc.reshape(NPAD * SL, LL), src, dst, zer2)
    ysc_n = _tc_layer(acc, ysc, dinv, b0, W1)
    acc = _mp_kernel(ysc_n.reshape(NPAD * SL, LL), src, dst, zer2)
    ysc_n2 = _tc_layer(acc, ysc_n, dinv, b1, W2)
    acc = _mp_kernel(ysc_n2.reshape(NPAD * SL, LL), src, dst, zer2)
    ysc_n3 = _tc_layer(acc, ysc_n2, dinv, b2, W3)
    acc = _mp_kernel(ysc_n3.reshape(NPAD * SL, LL), src, dst, zer2)
    h4 = _tc4(acc, ysc_n3, dinv, b3)

    mneg = jnp.full((MROWS,), -3.0e38, jnp.float32)
    zer_m = jnp.zeros((MROWS,), jnp.float32)
    maxp, sump, cntp = _pool_kernel(h4, batch_p, mneg, zer_m)
    return _head(maxp, sump, cntp, Wl, bl, Wo, bo)


# speedup vs baseline: 7.3767x; 3.1580x over previous
import functools

import jax
import jax.numpy as jnp
from jax import lax
from jax.experimental import pallas as pl
from jax.experimental.pallas import tpu as pltpu
from jax.experimental.pallas import tpu_sc as plsc

NN = 100000
EE = 1600000
GG = 128
HID = 128

NC, NS, LL = 2, 16, 16
SL = HID // LL
NPAD = 100352
NSTRIPE = NPAD // NS
CH = 128
NCHUNK = EE // CH
CTRIP = (NCHUNK + NS - 1) // NS

PSTRIPE = NPAD // (NC * NS)
MROWS = (GG + 1) * HID
CROWS = (GG + 1) * LL + 112

_mesh = plsc.VectorSubcoreMesh(core_axis_name="c", subcore_axis_name="s")


def _iota16():
    return lax.broadcasted_iota(jnp.int32, (LL,), 0)


@functools.partial(
    pl.kernel,
    out_type=[
        jax.ShapeDtypeStruct((NPAD,), jnp.float32),
        jax.ShapeDtypeStruct((NPAD,), jnp.float32),
    ],
    mesh=_mesh,
    scratch_types=[
        pltpu.VMEM_SHARED((NPAD,), jnp.float32),
        pltpu.VMEM((CH,), jnp.int32),
        pltpu.VMEM((CH,), jnp.float32),
    ],
)
def _deg_kernel(dst_hbm, zer_hbm, deg0_hbm, deg1_hbm, acc_sp, dst_v, ones_v):
    c = lax.axis_index("c")
    s = lax.axis_index("s")
    for j in range(CH // LL):
        ones_v[pl.ds(j * LL, LL)] = jnp.ones((LL,), jnp.float32)
    pltpu.sync_copy(zer_hbm.at[pl.ds(0, NSTRIPE)],
                    acc_sp.at[pl.ds(s * NSTRIPE, NSTRIPE)])
    plsc.subcore_barrier()

    def chunk(t, carry):
        k = c + NC * (s + NS * t)
        @pl.when(k < NCHUNK)
        def _():
            pltpu.sync_copy(dst_hbm.at[pl.ds(k * CH, CH)], dst_v)
            pltpu.sync_copy(ones_v, acc_sp.at[dst_v], add=True)
        return carry

    lax.fori_loop(0, (NCHUNK + NC * NS - 1) // (NC * NS), chunk, jnp.int32(0))
    plsc.subcore_barrier()

    @pl.when(c == 0)
    def _():
        pltpu.sync_copy(acc_sp.at[pl.ds(s * NSTRIPE, NSTRIPE)],
                        deg0_hbm.at[pl.ds(s * NSTRIPE, NSTRIPE)])

    @pl.when(c == 1)
    def _():
        pltpu.sync_copy(acc_sp.at[pl.ds(s * NSTRIPE, NSTRIPE)],
                        deg1_hbm.at[pl.ds(s * NSTRIPE, NSTRIPE)])


EB = 10
NBLK = NCHUNK // EB
BTRIP = (NBLK + NS - 1) // NS


@functools.partial(
    pl.kernel,
    out_type=jax.ShapeDtypeStruct((SL, NPAD, LL), jnp.float32),
    mesh=_mesh,
    scratch_types=[
        pltpu.VMEM_SHARED((NPAD, LL), jnp.float32),
        pltpu.VMEM((EB, CH), jnp.int32),
        pltpu.VMEM((EB, CH), jnp.int32),
        pltpu.VMEM((EB, CH), jnp.int32),
        pltpu.VMEM((EB, CH, LL), jnp.float32),
        pltpu.SemaphoreType.DMA,
        pltpu.SemaphoreType.DMA,
    ],
    compiler_params=pltpu.CompilerParams(use_tc_tiling_on_sc=False),
)
def _mp_kernel(ysc2_hbm, src2_hbm, dst2_hbm, zer2_hbm, out_hbm, acc_sp,
               src_v, sidx_v, dst_v, rows_v, semg, sems):
    c = lax.axis_index("c")
    s = lax.axis_index("s")

    def do_pass(p, carry):
        @pl.when(c == (p & 1))
        def _():
            pltpu.sync_copy(zer2_hbm,
                            acc_sp.at[pl.ds(s * NSTRIPE, NSTRIPE)])
            plsc.subcore_barrier()

            def block(u, carry2):
                blk = s + NS * u
                @pl.when(blk < NBLK)
                def _():
                    row0 = blk * EB
                    pltpu.sync_copy(src2_hbm.at[pl.ds(row0, EB)], src_v)
                    pltpu.sync_copy(dst2_hbm.at[pl.ds(row0, EB)], dst_v)
                    gd = []
                    for q in range(EB):
                        for j in range(CH // LL):
                            sv = src_v[q, pl.ds(j * LL, LL)]
                            sidx_v[q, pl.ds(j * LL, LL)] = sv * SL + p
                        gd.append(pltpu.async_copy(
                            ysc2_hbm.at[sidx_v.at[q]], rows_v.at[q], semg))
                    for d in gd:
                        d.wait()
                    sd = []
                    for q in range(EB):
                        sd.append(pltpu.async_copy(
                            rows_v.at[q], acc_sp.at[dst_v.at[q]], sems,
                            add=True))
                    for d in sd:
                        d.wait()
                return carry2

            lax.fori_loop(0, BTRIP, block, jnp.int32(0))
            plsc.subcore_barrier()
            pltpu.sync_copy(acc_sp.at[pl.ds(s * NSTRIPE, NSTRIPE)],
                            out_hbm.at[p, pl.ds(s * NSTRIPE, NSTRIPE)])
            plsc.subcore_barrier()
        return carry

    lax.fori_loop(0, SL, do_pass, jnp.int32(0))


@functools.partial(
    pl.kernel,
    out_type=[
        jax.ShapeDtypeStruct((NC * NS, MROWS), jnp.float32),
        jax.ShapeDtypeStruct((NC * NS, MROWS), jnp.float32),
        jax.ShapeDtypeStruct((NC * NS, CROWS), jnp.float32),
    ],
    mesh=_mesh,
    scratch_types=[
        pltpu.VMEM((MROWS,), jnp.float32),
        pltpu.VMEM((MROWS,), jnp.float32),
        pltpu.VMEM((CROWS,), jnp.float32),
        pltpu.VMEM((LL,), jnp.int32),
        pltpu.VMEM((LL, HID), jnp.float32),
    ],
    compiler_params=pltpu.CompilerParams(needs_layout_passes=False),
)
def _pool_kernel(h_hbm, batch_hbm, mneg_hbm, zer_hbm, maxout, sumout, cntout,
                 macc, sacc, cacc, b_v, rows_v):
    c = lax.axis_index("c")
    s = lax.axis_index("s")
    w = s * NC + c
    pltpu.sync_copy(mneg_hbm, macc)
    pltpu.sync_copy(zer_hbm, sacc)
    pltpu.sync_copy(zer_hbm.at[pl.ds(0, CROWS)], cacc)
    base = w * PSTRIPE
    ones = jnp.ones((LL,), jnp.float32)
    iot = _iota16()

    def step(t, carry):
        off = base + t * LL
        pltpu.sync_copy(batch_hbm.at[pl.ds(off, LL)], b_v)
        pltpu.sync_copy(h_hbm.at[pl.ds(off, LL)], rows_v)
        bv = b_v[...]
        plsc.addupdate_scatter(cacc, [bv * LL + iot], ones)
        for j in range(LL):
            bvj = plsc.load_gather(b_v, [jnp.full((LL,), j, jnp.int32)])
            for q in range(HID // LL):
                addr = bvj * HID + (q * LL + iot)
                r = rows_v[j, pl.ds(q * LL, LL)]
                m = plsc.load_gather(macc, [addr])
                plsc.store_scatter(macc, [addr], jnp.maximum(m, r))
                plsc.addupdate_scatter(sacc, [addr], r)
        return carry

    lax.fori_loop(0, PSTRIPE // LL, step, jnp.int32(0))
    pltpu.sync_copy(macc, maxout.at[w])
    pltpu.sync_copy(sacc, sumout.at[w])
    pltpu.sync_copy(cacc, cntout.at[w])


_BLK = 1024


def _tc0_body(x_ref, d0_ref, d1_ref, w_ref, ysc_ref, dinv_ref):
    d = lax.rsqrt(d0_ref[...] + d1_ref[...] + 1.0)
    y = jnp.dot(x_ref[...], w_ref[...], preferred_element_type=jnp.float32)
    ysc_ref[...] = y * d
    dinv_ref[...] = d


def _tc0(xp, deg0, deg1, W0):
    return pl.pallas_call(
        _tc0_body,
        grid=(NPAD // _BLK,),
        in_specs=[
            pl.BlockSpec((_BLK, 4), lambda i: (i, 0)),
            pl.BlockSpec((_BLK, 1), lambda i: (i, 0)),
            pl.BlockSpec((_BLK, 1), lambda i: (i, 0)),
            pl.BlockSpec((4, HID), lambda i: (0, 0)),
        ],
        out_specs=[
            pl.BlockSpec((_BLK, HID), lambda i: (i, 0)),
            pl.BlockSpec((_BLK, 1), lambda i: (i, 0)),
        ],
        out_shape=[
            jax.ShapeDtypeStruct((NPAD, HID), jnp.float32),
            jax.ShapeDtypeStruct((NPAD, 1), jnp.float32),
        ],
    )(xp, deg0, deg1, W0)


def _acc_concat(acc_ref):
    return jnp.concatenate([acc_ref[p] for p in range(SL)], axis=1)


def _tc_layer_body(acc_ref, ysc_ref, dinv_ref, b_ref, w_ref, out_ref):
    d = dinv_ref[...]
    agg = _acc_concat(acc_ref) + ysc_ref[...]
    h = jnp.tanh(agg * d + b_ref[...][None, :])
    y = jnp.dot(h, w_ref[...], preferred_element_type=jnp.float32)
    out_ref[...] = y * d


def _tc_layer(acc, ysc, dinv, b, W):
    return pl.pallas_call(
        _tc_layer_body,
        grid=(NPAD // _BLK,),
        in_specs=[
            pl.BlockSpec((SL, _BLK, LL), lambda i: (0, i, 0)),
            pl.BlockSpec((_BLK, HID), lambda i: (i, 0)),
            pl.BlockSpec((_BLK, 1), lambda i: (i, 0)),
            pl.BlockSpec((HID,), lambda i: (0,)),
            pl.BlockSpec((HID, HID), lambda i: (0, 0)),
        ],
        out_specs=pl.BlockSpec((_BLK, HID), lambda i: (i, 0)),
        out_shape=jax.ShapeDtypeStruct((NPAD, HID), jnp.float32),
    )(acc, ysc, dinv, b, W)


def _tc4_body(acc_ref, ysc_ref, dinv_ref, b_ref, out_ref):
    agg = _acc_concat(acc_ref) + ysc_ref[...]
    out_ref[...] = jnp.tanh(agg * dinv_ref[...] + b_ref[...][None, :])


def _tc4(acc, ysc, dinv, b):
    return pl.pallas_call(
        _tc4_body,
        grid=(NPAD // _BLK,),
        in_specs=[
            pl.BlockSpec((SL, _BLK, LL), lambda i: (0, i, 0)),
            pl.BlockSpec((_BLK, HID), lambda i: (i, 0)),
            pl.BlockSpec((_BLK, 1), lambda i: (i, 0)),
            pl.BlockSpec((HID,), lambda i: (0,)),
        ],
        out_specs=pl.BlockSpec((_BLK, HID), lambda i: (i, 0)),
        out_shape=jax.ShapeDtypeStruct((NPAD, HID), jnp.float32),
    )(acc, ysc, dinv, b)


_OUTW = 2145 * 2


def _head_body(maxp_ref, sump_ref, cntp_ref, wl_ref, bl_ref, wo_ref, bo_ref,
               out_ref, pooled_ref, cnt_ref):
    maxp = maxp_ref[...]
    sump = sump_ref[...]
    cntp = cntp_ref[...]
    for g in range(GG):
        mg = jnp.max(maxp[:, g * HID:(g + 1) * HID], axis=0)
        sg = jnp.sum(sump[:, g * HID:(g + 1) * HID], axis=0)
        cg = jnp.sum(cntp[:, g * LL:(g + 1) * LL], axis=0)
        pooled_ref[g] = jnp.concatenate([mg, sg])
        cnt_ref[g] = cg
    pm = pooled_ref[...]
    cv = jnp.sum(cnt_ref[...], axis=1, keepdims=True)
    fmax = jnp.where(cv > 0.0, 1.0, 0.0)
    fmean = 1.0 / jnp.maximum(cv, 1.0)
    pooled = jnp.concatenate(
        [pm[:, :HID] * fmax, pm[:, HID:] * fmean], axis=1)
    hid = jnp.dot(pooled, wl_ref[...],
                  preferred_element_type=jnp.float32) + bl_ref[...][None, :]
    out_ref[...] = jnp.dot(hid, wo_ref[...],
                           preferred_element_type=jnp.float32) + bo_ref[...][None, :]


def _head(maxp, sump, cntp, Wl, bl, Wo, bo):
    return pl.pallas_call(
        _head_body,
        out_shape=jax.ShapeDtypeStruct((GG, _OUTW), jnp.float32),
        scratch_shapes=[
            pltpu.VMEM((GG, 2 * HID), jnp.float32),
            pltpu.VMEM((GG, LL), jnp.float32),
        ],
    )(maxp, sump, cntp, Wl, bl, Wo, bo)


def kernel(x, edge_index, batch, W0, b0, W1, b1, W2, b2, W3, b3,
           Wl, bl, Wo, bo):
    src = edge_index[0].astype(jnp.int32)
    dst = edge_index[1].astype(jnp.int32)
    src2 = src.reshape(NCHUNK, CH)
    dst2 = dst.reshape(NCHUNK, CH)

    xp = jnp.pad(x, ((0, NPAD - NN), (0, 0)))
    batch_p = jnp.pad(batch.astype(jnp.int32), (0, NPAD - NN),
                      constant_values=GG)

    zer_n = jnp.zeros((NSTRIPE,), jnp.float32)
    zer2 = jnp.zeros((NSTRIPE, LL), jnp.float32)
    deg0, deg1 = _deg_kernel(dst, zer_n)

    ysc, dinv = _tc0(xp, deg0[:, None], deg1[:, None], W0)
    acc = _mp_kernel(ysc.reshape(NPAD * SL, LL), src2, dst2, zer2)
    ysc_n = _tc_layer(acc, ysc, dinv, b0, W1)
    acc = _mp_kernel(ysc_n.reshape(NPAD * SL, LL), src2, dst2, zer2)
    ysc_n2 = _tc_layer(acc, ysc_n, dinv, b1, W2)
    acc = _mp_kernel(ysc_n2.reshape(NPAD * SL, LL), src2, dst2, zer2)
    ysc_n3 = _tc_layer(acc, ysc_n2, dinv, b2, W3)
    acc = _mp_kernel(ysc_n3.reshape(NPAD * SL, LL), src2, dst2, zer2)
    h4 = _tc4(acc, ysc_n3, dinv, b3)

    mneg = jnp.full((MROWS,), -3.0e38, jnp.float32)
    zer_m = jnp.zeros((MROWS,), jnp.float32)
    maxp, sump, cntp = _pool_kernel(h4, batch_p, mneg, zer_m)
    return _head(maxp, sump, cntp, Wl, bl, Wo, bo)


# speedup vs baseline: 9.2632x; 1.2557x over previous
import functools

import jax
import jax.numpy as jnp
from jax import lax
from jax.experimental import pallas as pl
from jax.experimental.pallas import tpu as pltpu
from jax.experimental.pallas import tpu_sc as plsc

NN = 100000
EE = 1600000
GG = 128
HID = 128

NC, NS, LL = 2, 16, 16
SL = HID // LL
NPAD = 100352
NSTRIPE = NPAD // NS
CH = 128
NCHUNK = EE // CH
CTRIP = (NCHUNK + NS - 1) // NS

PSTRIPE = NPAD // (NC * NS)
MROWS = (GG + 1) * HID
CROWS = (GG + 1) * LL + 112

_mesh = plsc.VectorSubcoreMesh(core_axis_name="c", subcore_axis_name="s")


def _iota16():
    return lax.broadcasted_iota(jnp.int32, (LL,), 0)


DB = 25
DBLK = NCHUNK // NC // DB
DTRIP = (DBLK + NS - 1) // NS


@functools.partial(
    pl.kernel,
    out_type=[
        jax.ShapeDtypeStruct((NPAD,), jnp.float32),
        jax.ShapeDtypeStruct((NPAD,), jnp.float32),
    ],
    mesh=_mesh,
    scratch_types=[
        pltpu.VMEM_SHARED((NPAD,), jnp.float32),
        pltpu.VMEM((DB, CH), jnp.int32),
        pltpu.VMEM((CH,), jnp.float32),
        pltpu.SemaphoreType.DMA,
    ],
    compiler_params=pltpu.CompilerParams(use_tc_tiling_on_sc=False),
)
def _deg_kernel(dst2_hbm, zer_hbm, deg0_hbm, deg1_hbm, acc_sp, dst_v, ones_v,
                sems):
    c = lax.axis_index("c")
    s = lax.axis_index("s")
    for j in range(CH // LL):
        ones_v[pl.ds(j * LL, LL)] = jnp.ones((LL,), jnp.float32)
    pltpu.sync_copy(zer_hbm.at[pl.ds(0, NSTRIPE)],
                    acc_sp.at[pl.ds(s * NSTRIPE, NSTRIPE)])
    plsc.subcore_barrier()

    def block(u, carry):
        blk = s + NS * u
        @pl.when(blk < DBLK)
        def _():
            row0 = c * (NCHUNK // NC) + blk * DB
            pltpu.sync_copy(dst2_hbm.at[pl.ds(row0, DB)], dst_v)
            sd = []
            for q in range(DB):
                sd.append(pltpu.async_copy(
                    ones_v, acc_sp.at[dst_v.at[q]], sems, add=True))
            for d in sd:
                d.wait()
        return carry

    lax.fori_loop(0, DTRIP, block, jnp.int32(0))
    plsc.subcore_barrier()

    @pl.when(c == 0)
    def _():
        pltpu.sync_copy(acc_sp.at[pl.ds(s * NSTRIPE, NSTRIPE)],
                        deg0_hbm.at[pl.ds(s * NSTRIPE, NSTRIPE)])

    @pl.when(c == 1)
    def _():
        pltpu.sync_copy(acc_sp.at[pl.ds(s * NSTRIPE, NSTRIPE)],
                        deg1_hbm.at[pl.ds(s * NSTRIPE, NSTRIPE)])


EB = 10
NBLK = NCHUNK // EB
BTRIP = (NBLK + NS - 1) // NS


@functools.partial(
    pl.kernel,
    out_type=jax.ShapeDtypeStruct((SL, NPAD, LL), jnp.float32),
    mesh=_mesh,
    scratch_types=[
        pltpu.VMEM_SHARED((NPAD, LL), jnp.float32),
        pltpu.VMEM((EB, 2, CH), jnp.int32),
        pltpu.VMEM((EB, CH), jnp.int32),
        pltpu.VMEM((EB, CH, LL), jnp.float32),
        pltpu.SemaphoreType.DMA,
        pltpu.SemaphoreType.DMA,
        pltpu.SemaphoreType.DMA,
        pltpu.SemaphoreType.DMA,
        pltpu.SemaphoreType.DMA,
        pltpu.SemaphoreType.DMA,
    ],
    compiler_params=pltpu.CompilerParams(use_tc_tiling_on_sc=False),
)
def _mp_kernel(ysc2_hbm, sd2_hbm, zer2_hbm, out_hbm, acc_sp,
               sd_v, sidx_v, rows_v, semg0, semg1, semg2, semg3, semg4, sems):
    c = lax.axis_index("c")
    s = lax.axis_index("s")
    semg = [semg0, semg1, semg2, semg3, semg4]
    NG = len(semg)

    def fire_gather(q, p):
        for j in range(CH // LL):
            sv = sd_v[q, 0, pl.ds(j * LL, LL)]
            sidx_v[q, pl.ds(j * LL, LL)] = sv * SL + p
        return pltpu.async_copy(
            ysc2_hbm.at[sidx_v.at[q]], rows_v.at[q], semg[q % NG])

    def do_pass(p, carry):
        @pl.when(c == (p & 1))
        def _():
            pltpu.sync_copy(zer2_hbm,
                            acc_sp.at[pl.ds(s * NSTRIPE, NSTRIPE)])
            plsc.subcore_barrier()

            def block(u, carry2):
                blk = s + NS * u
                @pl.when(blk < NBLK)
                def _():
                    row0 = blk * EB
                    pltpu.sync_copy(sd2_hbm.at[pl.ds(row0, EB)], sd_v)
                    gd = [fire_gather(q, p) for q in range(NG)]
                    sd = []
                    for q in range(EB):
                        gd[q].wait()
                        sd.append(pltpu.async_copy(
                            rows_v.at[q], acc_sp.at[sd_v.at[q, 1]], sems,
                            add=True))
                        if q + NG < EB:
                            gd.append(fire_gather(q + NG, p))
                    for d in sd:
                        d.wait()
                return carry2

            lax.fori_loop(0, BTRIP, block, jnp.int32(0))
            plsc.subcore_barrier()
            pltpu.sync_copy(acc_sp.at[pl.ds(s * NSTRIPE, NSTRIPE)],
                            out_hbm.at[p, pl.ds(s * NSTRIPE, NSTRIPE)])
            plsc.subcore_barrier()
        return carry

    lax.fori_loop(0, SL, do_pass, jnp.int32(0))


@functools.partial(
    pl.kernel,
    out_type=[
        jax.ShapeDtypeStruct((NC * NS, MROWS), jnp.float32),
        jax.ShapeDtypeStruct((NC * NS, MROWS), jnp.float32),
        jax.ShapeDtypeStruct((NC * NS, CROWS), jnp.float32),
    ],
    mesh=_mesh,
    scratch_types=[
        pltpu.VMEM((MROWS,), jnp.float32),
        pltpu.VMEM((MROWS,), jnp.float32),
        pltpu.VMEM((CROWS,), jnp.float32),
        pltpu.VMEM((LL,), jnp.int32),
        pltpu.VMEM((LL, HID), jnp.float32),
    ],
    compiler_params=pltpu.CompilerParams(needs_layout_passes=False),
)
def _pool_kernel(h_hbm, batch_hbm, mneg_hbm, zer_hbm, maxout, sumout, cntout,
                 macc, sacc, cacc, b_v, rows_v):
    c = lax.axis_index("c")
    s = lax.axis_index("s")
    w = s * NC + c
    pltpu.sync_copy(mneg_hbm, macc)
    pltpu.sync_copy(zer_hbm, sacc)
    pltpu.sync_copy(zer_hbm.at[pl.ds(0, CROWS)], cacc)
    base = w * PSTRIPE
    ones = jnp.ones((LL,), jnp.float32)
    iot = _iota16()

    def step(t, carry):
        off = base + t * LL
        pltpu.sync_copy(batch_hbm.at[pl.ds(off, LL)], b_v)
        pltpu.sync_copy(h_hbm.at[pl.ds(off, LL)], rows_v)
        bv = b_v[...]
        plsc.addupdate_scatter(cacc, [bv * LL + iot], ones)
        for j in range(LL):
            bvj = plsc.load_gather(b_v, [jnp.full((LL,), j, jnp.int32)])
            for q in range(HID // LL):
                addr = bvj * HID + (q * LL + iot)
                r = rows_v[j, pl.ds(q * LL, LL)]
                m = plsc.load_gather(macc, [addr])
                plsc.store_scatter(macc, [addr], jnp.maximum(m, r))
                plsc.addupdate_scatter(sacc, [addr], r)
        return carry

    lax.fori_loop(0, PSTRIPE // LL, step, jnp.int32(0))
    pltpu.sync_copy(macc, maxout.at[w])
    pltpu.sync_copy(sacc, sumout.at[w])
    pltpu.sync_copy(cacc, cntout.at[w])


_BLK = 1024


def _tc0_body(x_ref, d0_ref, d1_ref, w_ref, ysc_ref, dinv_ref):
    d = lax.rsqrt(d0_ref[...] + d1_ref[...] + 1.0)
    y = jnp.dot(x_ref[...], w_ref[...], preferred_element_type=jnp.float32)
    ysc_ref[...] = y * d
    dinv_ref[...] = d


def _tc0(xp, deg0, deg1, W0):
    return pl.pallas_call(
        _tc0_body,
        grid=(NPAD // _BLK,),
        in_specs=[
            pl.BlockSpec((_BLK, 4), lambda i: (i, 0)),
            pl.BlockSpec((_BLK, 1), lambda i: (i, 0)),
            pl.BlockSpec((_BLK, 1), lambda i: (i, 0)),
            pl.BlockSpec((4, HID), lambda i: (0, 0)),
        ],
        out_specs=[
            pl.BlockSpec((_BLK, HID), lambda i: (i, 0)),
            pl.BlockSpec((_BLK, 1), lambda i: (i, 0)),
        ],
        out_shape=[
            jax.ShapeDtypeStruct((NPAD, HID), jnp.float32),
            jax.ShapeDtypeStruct((NPAD, 1), jnp.float32),
        ],
    )(xp, deg0, deg1, W0)


def _acc_concat(acc_ref):
    return jnp.concatenate([acc_ref[p] for p in range(SL)], axis=1)


def _tc_layer_body(acc_ref, ysc_ref, dinv_ref, b_ref, w_ref, out_ref):
    d = dinv_ref[...]
    agg = _acc_concat(acc_ref) + ysc_ref[...]
    h = jnp.tanh(agg * d + b_ref[...][None, :])
    y = jnp.dot(h, w_ref[...], preferred_element_type=jnp.float32)
    out_ref[...] = y * d


def _tc_layer(acc, ysc, dinv, b, W):
    return pl.pallas_call(
        _tc_layer_body,
        grid=(NPAD // _BLK,),
        in_specs=[
            pl.BlockSpec((SL, _BLK, LL), lambda i: (0, i, 0)),
            pl.BlockSpec((_BLK, HID), lambda i: (i, 0)),
            pl.BlockSpec((_BLK, 1), lambda i: (i, 0)),
            pl.BlockSpec((HID,), lambda i: (0,)),
            pl.BlockSpec((HID, HID), lambda i: (0, 0)),
        ],
        out_specs=pl.BlockSpec((_BLK, HID), lambda i: (i, 0)),
        out_shape=jax.ShapeDtypeStruct((NPAD, HID), jnp.float32),
    )(acc, ysc, dinv, b, W)


def _tc4_body(acc_ref, ysc_ref, dinv_ref, b_ref, out_ref):
    agg = _acc_concat(acc_ref) + ysc_ref[...]
    out_ref[...] = jnp.tanh(agg * dinv_ref[...] + b_ref[...][None, :])


def _tc4(acc, ysc, dinv, b):
    return pl.pallas_call(
        _tc4_body,
        grid=(NPAD // _BLK,),
        in_specs=[
            pl.BlockSpec((SL, _BLK, LL), lambda i: (0, i, 0)),
            pl.BlockSpec((_BLK, HID), lambda i: (i, 0)),
            pl.BlockSpec((_BLK, 1), lambda i: (i, 0)),
            pl.BlockSpec((HID,), lambda i: (0,)),
        ],
        out_specs=pl.BlockSpec((_BLK, HID), lambda i: (i, 0)),
        out_shape=jax.ShapeDtypeStruct((NPAD, HID), jnp.float32),
    )(acc, ysc, dinv, b)


_OUTW = 2145 * 2


def _head_body(maxp_ref, sump_ref, cntp_ref, wl_ref, bl_ref, wo_ref, bo_ref,
               out_ref, pooled_ref, cnt_ref):
    maxp = maxp_ref[...]
    sump = sump_ref[...]
    cntp = cntp_ref[...]
    for g in range(GG):
        mg = jnp.max(maxp[:, g * HID:(g + 1) * HID], axis=0)
        sg = jnp.sum(sump[:, g * HID:(g + 1) * HID], axis=0)
        cg = jnp.sum(cntp[:, g * LL:(g + 1) * LL], axis=0)
        pooled_ref[g] = jnp.concatenate([mg, sg])
        cnt_ref[g] = cg
    pm = pooled_ref[...]
    cv = jnp.sum(cnt_ref[...], axis=1, keepdims=True)
    fmax = jnp.where(cv > 0.0, 1.0, 0.0)
    fmean = 1.0 / jnp.maximum(cv, 1.0)
    pooled = jnp.concatenate(
        [pm[:, :HID] * fmax, pm[:, HID:] * fmean], axis=1)
    hid = jnp.dot(pooled, wl_ref[...],
                  preferred_element_type=jnp.float32) + bl_ref[...][None, :]
    out_ref[...] = jnp.dot(hid, wo_ref[...],
                           preferred_element_type=jnp.float32) + bo_ref[...][None, :]


def _head(maxp, sump, cntp, Wl, bl, Wo, bo):
    return pl.pallas_call(
        _head_body,
        out_shape=jax.ShapeDtypeStruct((GG, _OUTW), jnp.float32),
        scratch_shapes=[
            pltpu.VMEM((GG, 2 * HID), jnp.float32),
            pltpu.VMEM((GG, LL), jnp.float32),
        ],
    )(maxp, sump, cntp, Wl, bl, Wo, bo)


def kernel(x, edge_index, batch, W0, b0, W1, b1, W2, b2, W3, b3,
           Wl, bl, Wo, bo):
    src = edge_index[0].astype(jnp.int32)
    dst = edge_index[1].astype(jnp.int32)
    src2 = src.reshape(NCHUNK, CH)
    dst2 = dst.reshape(NCHUNK, CH)
    sd2 = jnp.stack([src2, dst2], axis=1)

    xp = jnp.pad(x, ((0, NPAD - NN), (0, 0)))
    batch_p = jnp.pad(batch.astype(jnp.int32), (0, NPAD - NN),
                      constant_values=GG)

    zer_n = jnp.zeros((NSTRIPE,), jnp.float32)
    zer2 = jnp.zeros((NSTRIPE, LL), jnp.float32)
    deg0, deg1 = _deg_kernel(dst2, zer_n)

    ysc, dinv = _tc0(xp, deg0[:, None], deg1[:, None], W0)
    acc = _mp_kernel(ysc.reshape(NPAD * SL, LL), sd2, zer2)
    ysc_n = _tc_layer(acc, ysc, dinv, b0, W1)
    acc = _mp_kernel(ysc_n.reshape(NPAD * SL, LL), sd2, zer2)
    ysc_n2 = _tc_layer(acc, ysc_n, dinv, b1, W2)
    acc = _mp_kernel(ysc_n2.reshape(NPAD * SL, LL), sd2, zer2)
    ysc_n3 = _tc_layer(acc, ysc_n2, dinv, b2, W3)
    acc = _mp_kernel(ysc_n3.reshape(NPAD * SL, LL), sd2, zer2)
    h4 = _tc4(acc, ysc_n3, dinv, b3)

    mneg = jnp.full((MROWS,), -3.0e38, jnp.float32)
    zer_m = jnp.zeros((MROWS,), jnp.float32)
    maxp, sump, cntp = _pool_kernel(h4, batch_p, mneg, zer_m)
    return _head(maxp, sump, cntp, Wl, bl, Wo, bo)


# speedup vs baseline: 9.7868x; 1.0565x over previous
import functools

import jax
import jax.numpy as jnp
from jax import lax
from jax.experimental import pallas as pl
from jax.experimental.pallas import tpu as pltpu
from jax.experimental.pallas import tpu_sc as plsc

NN = 100000
EE = 1600000
GG = 128
HID = 128

NC, NS, LL = 2, 16, 16
SL = HID // LL
NPAD = 100352
NSTRIPE = NPAD // NS
CH = 128
NCHUNK = EE // CH
CTRIP = (NCHUNK + NS - 1) // NS

PSTRIPE = NPAD // (NC * NS)
PCH = 32
MROWS = (GG + 1) * HID
CROWS = (GG + 1) * LL + 112

_mesh = plsc.VectorSubcoreMesh(core_axis_name="c", subcore_axis_name="s")


def _iota16():
    return lax.broadcasted_iota(jnp.int32, (LL,), 0)


DB = 25
DBLK = NCHUNK // NC // DB
DTRIP = (DBLK + NS - 1) // NS


@functools.partial(
    pl.kernel,
    out_type=[
        jax.ShapeDtypeStruct((NPAD,), jnp.float32),
        jax.ShapeDtypeStruct((NPAD,), jnp.float32),
    ],
    mesh=_mesh,
    scratch_types=[
        pltpu.VMEM_SHARED((NPAD,), jnp.float32),
        pltpu.VMEM((DB, CH), jnp.int32),
        pltpu.VMEM((CH,), jnp.float32),
        pltpu.SemaphoreType.DMA,
    ],
    compiler_params=pltpu.CompilerParams(use_tc_tiling_on_sc=False),
)
def _deg_kernel(dst2_hbm, zer_hbm, deg0_hbm, deg1_hbm, acc_sp, dst_v, ones_v,
                sems):
    c = lax.axis_index("c")
    s = lax.axis_index("s")
    for j in range(CH // LL):
        ones_v[pl.ds(j * LL, LL)] = jnp.ones((LL,), jnp.float32)
    pltpu.sync_copy(zer_hbm.at[pl.ds(0, NSTRIPE)],
                    acc_sp.at[pl.ds(s * NSTRIPE, NSTRIPE)])
    plsc.subcore_barrier()

    def block(u, carry):
        blk = s + NS * u
        @pl.when(blk < DBLK)
        def _():
            row0 = c * (NCHUNK // NC) + blk * DB
            pltpu.sync_copy(dst2_hbm.at[pl.ds(row0, DB)], dst_v)
            sd = []
            for q in range(DB):
                sd.append(pltpu.async_copy(
                    ones_v, acc_sp.at[dst_v.at[q]], sems, add=True))
            for d in sd:
                d.wait()
        return carry

    lax.fori_loop(0, DTRIP, block, jnp.int32(0))
    plsc.subcore_barrier()

    @pl.when(c == 0)
    def _():
        pltpu.sync_copy(acc_sp.at[pl.ds(s * NSTRIPE, NSTRIPE)],
                        deg0_hbm.at[pl.ds(s * NSTRIPE, NSTRIPE)])

    @pl.when(c == 1)
    def _():
        pltpu.sync_copy(acc_sp.at[pl.ds(s * NSTRIPE, NSTRIPE)],
                        deg1_hbm.at[pl.ds(s * NSTRIPE, NSTRIPE)])


EB = 10
NBLK = NCHUNK // EB
BTRIP = (NBLK + NS - 1) // NS


@functools.partial(
    pl.kernel,
    out_type=jax.ShapeDtypeStruct((SL, NPAD, LL), jnp.float32),
    mesh=_mesh,
    scratch_types=[
        pltpu.VMEM_SHARED((NPAD, LL), jnp.float32),
        pltpu.VMEM((EB, 2, CH), jnp.int32),
        pltpu.VMEM((EB, CH), jnp.int32),
        pltpu.VMEM((EB, CH, LL), jnp.float32),
        pltpu.SemaphoreType.DMA,
    ] + [pltpu.SemaphoreType.DMA] * EB,
    compiler_params=pltpu.CompilerParams(use_tc_tiling_on_sc=False),
)
def _mp_kernel(ysc2_hbm, sd2_hbm, zer2_hbm, out_hbm, acc_sp,
               sd_v, sidx_v, rows_v, sems, *semg):
    c = lax.axis_index("c")
    s = lax.axis_index("s")

    def fire_gather(q, p):
        for j in range(CH // LL):
            sv = sd_v[q, 0, pl.ds(j * LL, LL)]
            sidx_v[q, pl.ds(j * LL, LL)] = sv * SL + p
        return pltpu.async_copy(
            ysc2_hbm.at[sidx_v.at[q]], rows_v.at[q], semg[q])

    def do_pass(p, carry):
        @pl.when(c == (p & 1))
        def _():
            pltpu.sync_copy(zer2_hbm,
                            acc_sp.at[pl.ds(s * NSTRIPE, NSTRIPE)])
            plsc.subcore_barrier()

            def block(u, carry2):
                blk = s + NS * u
                @pl.when(blk < NBLK)
                def _():
                    row0 = blk * EB
                    pltpu.sync_copy(sd2_hbm.at[pl.ds(row0, EB)], sd_v)
                    gd = [fire_gather(q, p) for q in range(EB)]
                    sdl = []
                    for q in range(EB):
                        gd[q].wait()
                        sdl.append(pltpu.async_copy(
                            rows_v.at[q], acc_sp.at[sd_v.at[q, 1]], sems,
                            add=True))
                    for d in sdl:
                        d.wait()
                return carry2

            lax.fori_loop(0, BTRIP, block, jnp.int32(0))
            plsc.subcore_barrier()
            pltpu.sync_copy(acc_sp.at[pl.ds(s * NSTRIPE, NSTRIPE)],
                            out_hbm.at[p, pl.ds(s * NSTRIPE, NSTRIPE)])
            plsc.subcore_barrier()
        return carry

    lax.fori_loop(0, SL, do_pass, jnp.int32(0))


@functools.partial(
    pl.kernel,
    out_type=[
        jax.ShapeDtypeStruct((NC * NS, MROWS), jnp.float32),
        jax.ShapeDtypeStruct((NC * NS, MROWS), jnp.float32),
        jax.ShapeDtypeStruct((NC * NS, CROWS), jnp.float32),
    ],
    mesh=_mesh,
    scratch_types=[
        pltpu.VMEM((MROWS,), jnp.float32),
        pltpu.VMEM((MROWS,), jnp.float32),
        pltpu.VMEM((CROWS,), jnp.float32),
        pltpu.VMEM((LL,), jnp.int32),
        pltpu.VMEM((LL, HID), jnp.float32),
    ],
    compiler_params=pltpu.CompilerParams(needs_layout_passes=False),
)
def _pool_kernel(h_hbm, batch_hbm, mneg_hbm, zer_hbm, maxout, sumout, cntout,
                 macc, sacc, cacc, b_v, rows_v):
    c = lax.axis_index("c")
    s = lax.axis_index("s")
    w = s * NC + c
    pltpu.sync_copy(mneg_hbm, macc)
    pltpu.sync_copy(zer_hbm, sacc)
    pltpu.sync_copy(zer_hbm.at[pl.ds(0, CROWS)], cacc)
    base = w * PSTRIPE
    ones = jnp.ones((LL,), jnp.float32)
    iot = _iota16()

    def step(t, carry):
        off = base + t * LL
        pltpu.sync_copy(batch_hbm.at[pl.ds(off, LL)], b_v)
        pltpu.sync_copy(h_hbm.at[pl.ds(off, LL)], rows_v)
        bv = b_v[...]
        plsc.addupdate_scatter(cacc, [bv * LL + iot], ones)
        for j in range(LL):
            bvj = plsc.load_gather(b_v, [jnp.full((LL,), j, jnp.int32)])
            for q in range(HID // LL):
                addr = bvj * HID + (q * LL + iot)
                r = rows_v[j, pl.ds(q * LL, LL)]
                m = plsc.load_gather(macc, [addr])
                plsc.store_scatter(macc, [addr], jnp.maximum(m, r))
                plsc.addupdate_scatter(sacc, [addr], r)
        return carry

    lax.fori_loop(0, PSTRIPE // LL, step, jnp.int32(0))
    pltpu.sync_copy(macc, maxout.at[w])
    pltpu.sync_copy(sacc, sumout.at[w])
    pltpu.sync_copy(cacc, cntout.at[w])


_BLK = 1024


def _tc0_body(x_ref, d0_ref, d1_ref, w_ref, ysc_ref, dinv_ref):
    d = lax.rsqrt(d0_ref[...] + d1_ref[...] + 1.0)
    y = jnp.dot(x_ref[...], w_ref[...], preferred_element_type=jnp.float32)
    ysc_ref[...] = y * d
    dinv_ref[...] = d


def _tc0(xp, deg0, deg1, W0):
    return pl.pallas_call(
        _tc0_body,
        grid=(NPAD // _BLK,),
        in_specs=[
            pl.BlockSpec((_BLK, 4), lambda i: (i, 0)),
            pl.BlockSpec((_BLK, 1), lambda i: (i, 0)),
            pl.BlockSpec((_BLK, 1), lambda i: (i, 0)),
            pl.BlockSpec((4, HID), lambda i: (0, 0)),
        ],
        out_specs=[
            pl.BlockSpec((_BLK, HID), lambda i: (i, 0)),
            pl.BlockSpec((_BLK, 1), lambda i: (i, 0)),
        ],
        out_shape=[
            jax.ShapeDtypeStruct((NPAD, HID), jnp.float32),
            jax.ShapeDtypeStruct((NPAD, 1), jnp.float32),
        ],
    )(xp, deg0, deg1, W0)


def _acc_concat(acc_ref):
    return jnp.concatenate([acc_ref[p] for p in range(SL)], axis=1)


def _tc_layer_body(acc_ref, ysc_ref, dinv_ref, b_ref, w_ref, out_ref):
    d = dinv_ref[...]
    agg = _acc_concat(acc_ref) + ysc_ref[...]
    h = jnp.tanh(agg * d + b_ref[...][None, :])
    y = jnp.dot(h, w_ref[...], preferred_element_type=jnp.float32)
    out_ref[...] = y * d


def _tc_layer(acc, ysc, dinv, b, W):
    return pl.pallas_call(
        _tc_layer_body,
        grid=(NPAD // _BLK,),
        in_specs=[
            pl.BlockSpec((SL, _BLK, LL), lambda i: (0, i, 0)),
            pl.BlockSpec((_BLK, HID), lambda i: (i, 0)),
            pl.BlockSpec((_BLK, 1), lambda i: (i, 0)),
            pl.BlockSpec((HID,), lambda i: (0,)),
            pl.BlockSpec((HID, HID), lambda i: (0, 0)),
        ],
        out_specs=pl.BlockSpec((_BLK, HID), lambda i: (i, 0)),
        out_shape=jax.ShapeDtypeStruct((NPAD, HID), jnp.float32),
    )(acc, ysc, dinv, b, W)


def _tc4_body(acc_ref, ysc_ref, dinv_ref, b_ref, out_ref):
    agg = _acc_concat(acc_ref) + ysc_ref[...]
    out_ref[...] = jnp.tanh(agg * dinv_ref[...] + b_ref[...][None, :])


def _tc4(acc, ysc, dinv, b):
    return pl.pallas_call(
        _tc4_body,
        grid=(NPAD // _BLK,),
        in_specs=[
            pl.BlockSpec((SL, _BLK, LL), lambda i: (0, i, 0)),
            pl.BlockSpec((_BLK, HID), lambda i: (i, 0)),
            pl.BlockSpec((_BLK, 1), lambda i: (i, 0)),
            pl.BlockSpec((HID,), lambda i: (0,)),
        ],
        out_specs=pl.BlockSpec((_BLK, HID), lambda i: (i, 0)),
        out_shape=jax.ShapeDtypeStruct((NPAD, HID), jnp.float32),
    )(acc, ysc, dinv, b)


_OUTW = 2145 * 2


def _head_body(maxp_ref, sump_ref, cntp_ref, wl_ref, bl_ref, wo_ref, bo_ref,
               out_ref, pooled_ref, cnt_ref):
    maxp = maxp_ref[...]
    sump = sump_ref[...]
    cntp = cntp_ref[...]
    for g in range(GG):
        mg = jnp.max(maxp[:, g * HID:(g + 1) * HID], axis=0)
        sg = jnp.sum(sump[:, g * HID:(g + 1) * HID], axis=0)
        cg = jnp.sum(cntp[:, g * LL:(g + 1) * LL], axis=0)
        pooled_ref[g] = jnp.concatenate([mg, sg])
        cnt_ref[g] = cg
    pm = pooled_ref[...]
    cv = jnp.sum(cnt_ref[...], axis=1, keepdims=True)
    fmax = jnp.where(cv > 0.0, 1.0, 0.0)
    fmean = 1.0 / jnp.maximum(cv, 1.0)
    pooled = jnp.concatenate(
        [pm[:, :HID] * fmax, pm[:, HID:] * fmean], axis=1)
    hid = jnp.dot(pooled, wl_ref[...],
                  preferred_element_type=jnp.float32) + bl_ref[...][None, :]
    out_ref[...] = jnp.dot(hid, wo_ref[...],
                           preferred_element_type=jnp.float32) + bo_ref[...][None, :]


def _head(maxp, sump, cntp, Wl, bl, Wo, bo):
    return pl.pallas_call(
        _head_body,
        out_shape=jax.ShapeDtypeStruct((GG, _OUTW), jnp.float32),
        scratch_shapes=[
            pltpu.VMEM((GG, 2 * HID), jnp.float32),
            pltpu.VMEM((GG, LL), jnp.float32),
        ],
    )(maxp, sump, cntp, Wl, bl, Wo, bo)


def kernel(x, edge_index, batch, W0, b0, W1, b1, W2, b2, W3, b3,
           Wl, bl, Wo, bo):
    src = edge_index[0].astype(jnp.int32)
    dst = edge_index[1].astype(jnp.int32)
    src2 = src.reshape(NCHUNK, CH)
    dst2 = dst.reshape(NCHUNK, CH)
    sd2 = jnp.stack([src2, dst2], axis=1)

    xp = jnp.pad(x, ((0, NPAD - NN), (0, 0)))
    batch_p = jnp.pad(batch.astype(jnp.int32), (0, NPAD - NN),
                      constant_values=GG)

    zer_n = jnp.zeros((NSTRIPE,), jnp.float32)
    zer2 = jnp.zeros((NSTRIPE, LL), jnp.float32)
    deg0, deg1 = _deg_kernel(dst2, zer_n)

    ysc, dinv = _tc0(xp, deg0[:, None], deg1[:, None], W0)
    acc = _mp_kernel(ysc.reshape(NPAD * SL, LL), sd2, zer2)
    ysc_n = _tc_layer(acc, ysc, dinv, b0, W1)
    acc = _mp_kernel(ysc_n.reshape(NPAD * SL, LL), sd2, zer2)
    ysc_n2 = _tc_layer(acc, ysc_n, dinv, b1, W2)
    acc = _mp_kernel(ysc_n2.reshape(NPAD * SL, LL), sd2, zer2)
    ysc_n3 = _tc_layer(acc, ysc_n2, dinv, b2, W3)
    acc = _mp_kernel(ysc_n3.reshape(NPAD * SL, LL), sd2, zer2)
    h4 = _tc4(acc, ysc_n3, dinv, b3)

    mneg = jnp.full((MROWS,), -3.0e38, jnp.float32)
    zer_m = jnp.zeros((MROWS,), jnp.float32)
    maxp, sump, cntp = _pool_kernel(h4, batch_p, mneg, zer_m)
    return _head(maxp, sump, cntp, Wl, bl, Wo, bo)
